# baseline (speedup 1.0000x reference)
import functools

import jax
import jax.numpy as jnp
from jax import lax
from jax.experimental import pallas as pl
from jax.experimental.pallas import tpu as pltpu
from jax.experimental.pallas import tpu_sc as plsc

N = 10000
F = 128
NC, NS = 2, 16
NW = NC * NS
E = 320000
E_PER_W = E // NW
CH = 64
NCHUNK = 160
EPAD = NCHUNK * CH
NDUMMY = N
NACC = N + 8
NBUF = 4
GSHIFT = 17
BN = 2000


def _full(shape):
    zeros = (0,) * len(shape)
    return pl.BlockSpec(shape, lambda *_: zeros)


def _transform_body(h_ref, wt_ref, b_ref, out_ref):
    out_ref[...] = (
        jnp.dot(h_ref[...], wt_ref[0], preferred_element_type=jnp.float32)
        + b_ref[0]
    )


def _transform(h, WT, b, T):
    nb = N // BN
    return pl.pallas_call(
        _transform_body,
        grid=(nb, T),
        in_specs=[
            pl.BlockSpec((BN, F), lambda i, t: (i, 0)),
            pl.BlockSpec((1, F, F), lambda i, t: (t, 0, 0)),
            pl.BlockSpec((1, 1, F), lambda i, t: (t, 0, 0)),
        ],
        out_specs=pl.BlockSpec((BN, F), lambda i, t: (t * (N // BN) + i, 0)),
        out_shape=jax.ShapeDtypeStruct((T * N, F), jnp.float32),
    )(h, WT, b.reshape(T, 1, F))


def _gru_math(a2, h, wihT, whhT, bih, bhh):
    a = a2[0] + a2[1]
    gi = jnp.dot(a, wihT, preferred_element_type=jnp.float32) + bih
    gh = jnp.dot(h, whhT, preferred_element_type=jnp.float32) + bhh
    r = jax.nn.sigmoid(gi[:, :F] + gh[:, :F])
    z = jax.nn.sigmoid(gi[:, F:2 * F] + gh[:, F:2 * F])
    n = jnp.tanh(gi[:, 2 * F:] + r * gh[:, 2 * F:])
    return (1.0 - z) * n + z * h


def _gate_math(h, f, wiaT, wibT, bi, wjT, bj):
    g = (
        jnp.dot(h, wiaT, preferred_element_type=jnp.float32)
        + jnp.dot(f, wibT, preferred_element_type=jnp.float32)
        + bi
    )
    return jax.nn.sigmoid(g) * (
        jnp.dot(h, wjT, preferred_element_type=jnp.float32) + bj
    )


def _gru_tf(a2, h, WihT, WhhT, bih, bhh, WT, b, T, bn):
    nb = N // bn

    def body(a_ref, h_ref, wihT_ref, whhT_ref, bih_ref, bhh_ref, wT_ref,
             b_ref, hout_ref, tout_ref):
        hnew = _gru_math(a_ref[...], h_ref[...], wihT_ref[...], whhT_ref[...],
                         bih_ref[...], bhh_ref[...])
        hout_ref[...] = hnew
        for t in range(T):
            tout_ref[t] = (
                jnp.dot(hnew, wT_ref[t], preferred_element_type=jnp.float32)
                + b_ref[t]
            )

    hout, tout = pl.pallas_call(
        body,
        grid=(nb,),
        in_specs=[
            pl.BlockSpec((2, bn, F), lambda i: (0, i, 0)),
            pl.BlockSpec((bn, F), lambda i: (i, 0)),
            _full((F, 3 * F)),
            _full((F, 3 * F)),
            _full((1, 3 * F)),
            _full((1, 3 * F)),
            _full((T, F, F)),
            _full((T, 1, F)),
        ],
        out_specs=[
            pl.BlockSpec((bn, F), lambda i: (i, 0)),
            pl.BlockSpec((T, bn, F), lambda i: (0, i, 0)),
        ],
        out_shape=[
            jax.ShapeDtypeStruct((N, F), jnp.float32),
            jax.ShapeDtypeStruct((T, N, F), jnp.float32),
        ],
    )(a2, h, WihT, WhhT, bih, bhh, WT, b.reshape(T, 1, F))
    return hout, tout.reshape(T * N, F)


def _gru_gate_tf(a2, h, feat, WihT, WhhT, bih, bhh,
                 WiaT, WibT, bi, WjT, bj, WT, b):
    nb = N // BN

    def body(a_ref, h_ref, f_ref, wihT_ref, whhT_ref, bih_ref, bhh_ref,
             wiaT_ref, wibT_ref, bi_ref, wjT_ref, bj_ref, wT_ref, b_ref,
             hout_ref, tout_ref):
        hnew = _gru_math(a_ref[...], h_ref[...], wihT_ref[...], whhT_ref[...],
                         bih_ref[...], bhh_ref[...])
        hg = _gate_math(hnew, f_ref[...], wiaT_ref[...], wibT_ref[...],
                        bi_ref[...], wjT_ref[...], bj_ref[...])
        hout_ref[...] = hg
        tout_ref[...] = (
            jnp.dot(hg, wT_ref[0], preferred_element_type=jnp.float32)
            + b_ref[0]
        )

    hout, tout = pl.pallas_call(
        body,
        grid=(nb,),
        in_specs=[
            pl.BlockSpec((2, BN, F), lambda i: (0, i, 0)),
            pl.BlockSpec((BN, F), lambda i: (i, 0)),
            pl.BlockSpec((BN, F), lambda i: (i, 0)),
            _full((F, 3 * F)),
            _full((F, 3 * F)),
            _full((1, 3 * F)),
            _full((1, 3 * F)),
            _full((F, F)),
            _full((F, F)),
            _full((1, F)),
            _full((F, F)),
            _full((1, F)),
            _full((1, F, F)),
            _full((1, 1, F)),
        ],
        out_specs=[
            pl.BlockSpec((BN, F), lambda i: (i, 0)),
            pl.BlockSpec((BN, F), lambda i: (i, 0)),
        ],
        out_shape=[
            jax.ShapeDtypeStruct((N, F), jnp.float32),
            jax.ShapeDtypeStruct((N, F), jnp.float32),
        ],
    )(a2, h, feat, WihT, WhhT, bih, bhh, WiaT, WibT, bi, WjT, bj,
      WT, b.reshape(1, 1, F))
    return hout, tout


def _gru_gate_readout(a2, h, feat, WihT, WhhT, bih, bhh,
                      WiaT, WibT, bi, WjT, bj, na,
                      Wf0T, bf0, Wf1T, bf1, WoutT, bout):
    nb = N // BN

    def body(a_ref, h_ref, f_ref, wihT_ref, whhT_ref, bih_ref, bhh_ref,
             wiaT_ref, wibT_ref, bi_ref, wjT_ref, bj_ref, na_ref,
             wf0T_ref, bf0_ref, wf1T_ref, bf1_ref, woutT_ref, bout_ref,
             out_ref, acc_ref):
        i = pl.program_id(0)

        @pl.when(i == 0)
        def _():
            acc_ref[...] = jnp.zeros_like(acc_ref)

        hnew = _gru_math(a_ref[...], h_ref[...], wihT_ref[...], whhT_ref[...],
                         bih_ref[...], bhh_ref[...])
        hg = _gate_math(hnew, f_ref[...], wiaT_ref[...], wibT_ref[...],
                        bi_ref[...], wjT_ref[...], bj_ref[...])
        rows = jax.lax.broadcasted_iota(jnp.int32, (BN, 1), 0) + i * BN
        masked = jnp.where(rows < na_ref[0], hg, 0.0)
        acc_ref[...] += jnp.sum(masked.reshape(BN // 8, 8, F), axis=0)

        @pl.when(i == pl.num_programs(0) - 1)
        def _():
            hp = jax.lax.Precision.HIGHEST
            v = jnp.sum(acc_ref[...], axis=0, keepdims=True)
            v = jax.nn.relu(
                jnp.dot(v, wf0T_ref[...], preferred_element_type=jnp.float32,
                        precision=hp) + bf0_ref[...])
            v = jax.nn.relu(
                jnp.dot(v, wf1T_ref[...], preferred_element_type=jnp.float32,
                        precision=hp) + bf1_ref[...])
            out_ref[...] = (
                jnp.dot(v, woutT_ref[...], preferred_element_type=jnp.float32,
                        precision=hp) + bout_ref[...])

    return pl.pallas_call(
        body,
        grid=(nb,),
        in_specs=[
            pl.BlockSpec((2, BN, F), lambda i: (0, i, 0)),
            pl.BlockSpec((BN, F), lambda i: (i, 0)),
            pl.BlockSpec((BN, F), lambda i: (i, 0)),
            _full((F, 3 * F)),
            _full((F, 3 * F)),
            _full((1, 3 * F)),
            _full((1, 3 * F)),
            _full((F, F)),
            _full((F, F)),
            _full((1, F)),
            _full((F, F)),
            _full((1, F)),
            pl.BlockSpec(memory_space=pltpu.SMEM),
            _full((F, F)),
            _full((1, F)),
            _full((F, F)),
            _full((1, F)),
            _full((F, 1)),
            _full((1, 1)),
        ],
        out_specs=pl.BlockSpec((1, 1), lambda i: (0, 0)),
        out_shape=jax.ShapeDtypeStruct((1, 1), jnp.float32),
        scratch_shapes=[pltpu.VMEM((8, F), jnp.float32)],
    )(a2, h, feat, WihT, WhhT, bih, bhh, WiaT, WibT, bi, WjT, bj, na,
      Wf0T, bf0, Wf1T, bf1, WoutT, bout)


@functools.lru_cache(maxsize=None)
def _make_gs(t_rows):
    mesh = plsc.VectorSubcoreMesh(
        core_axis_name="c", subcore_axis_name="s", num_cores=NC, num_subcores=NS)

    @functools.partial(
        pl.kernel, mesh=mesh,
        out_type=jax.ShapeDtypeStruct((NC, N, F), jnp.float32),
        scratch_types=(
            [
                pltpu.VMEM((EPAD,), jnp.int32),
                pltpu.VMEM((NBUF, CH), jnp.int32),
                pltpu.VMEM((NBUF, CH), jnp.int32),
                pltpu.VMEM((NBUF, CH, F), jnp.float32),
                pltpu.VMEM_SHARED((NACC, F), jnp.float32),
            ]
            + [pltpu.SemaphoreType.DMA] * (2 * NBUF)
        ),
    )
    def gs(table, pidx, zeros, out, p_v, gi_v, di_v, rows_v, acc, *sems):
        sem_g = sems[:NBUF]
        sem_s = sems[NBUF:]
        cid = lax.axis_index("c")
        sid = lax.axis_index("s")
        wid = sid * NC + cid
        base = pl.multiple_of(sid * 624, 8)

        @pl.when(sid < NS - 1)
        def _():
            pltpu.sync_copy(zeros.at[pl.ds(base, 624)], acc.at[pl.ds(base, 624)])

        @pl.when(sid == NS - 1)
        def _():
            pltpu.sync_copy(zeros.at[pl.ds(base, 648)], acc.at[pl.ds(base, 648)])

        pltpu.sync_copy(pidx.at[wid], p_v)
        plsc.subcore_barrier()

        gmask = jnp.full((16,), (1 << GSHIFT) - 1, jnp.int32)

        def unpack(b, c):
            for k in range(CH // 16):
                v = p_v[pl.ds(c * CH + k * 16, 16)]
                gi_v[b, pl.ds(k * 16, 16)] = v & gmask
                di_v[b, pl.ds(k * 16, 16)] = lax.shift_right_logical(v, GSHIFT)

        def gather_start(b, c):
            pltpu.async_copy(table.at[gi_v.at[b]], rows_v.at[b], sem_g[b])

        def gather_wait(b):
            pltpu.make_async_copy(table.at[gi_v.at[b]], rows_v.at[b],
                                  sem_g[b]).wait()

        def scatter_start(b):
            pltpu.async_copy(rows_v.at[b], acc.at[di_v.at[b]], sem_s[b],
                             add=True)

        def scatter_wait(b):
            pltpu.make_async_copy(rows_v.at[b], acc.at[di_v.at[b]],
                                  sem_s[b]).wait()

        for c in range(NBUF - 1):
            unpack(c, c)
            gather_start(c, c)

        def round_body(r, carry):
            for s in range(NBUF):
                c = r * NBUF + s
                b_pre = (s + NBUF - 1) % NBUF

                @pl.when(c >= 1)
                def _():
                    scatter_wait(b_pre)

                @pl.when(c + NBUF - 1 < NCHUNK)
                def _():
                    unpack(b_pre, c + NBUF - 1)
                    gather_start(b_pre, c + NBUF - 1)

                gather_wait(s)
                scatter_start(s)
            return carry

        lax.fori_loop(0, NCHUNK // NBUF, round_body, 0)
        scatter_wait((NCHUNK - 1) % NBUF)
        plsc.subcore_barrier()

        @pl.when(sid < NS - 1)
        def _():
            pltpu.sync_copy(acc.at[pl.ds(base, 624)],
                            out.at[cid, pl.ds(base, 624)])

        @pl.when(sid == NS - 1)
        def _():
            pltpu.sync_copy(acc.at[pl.ds(base, 640)],
                            out.at[cid, pl.ds(base, 640)])

    return gs


def _gs_call(table, pidx, zeros):
    return _make_gs(table.shape[0])(table, pidx, zeros)


def kernel(x, W1, b1, Wih1, Whh1, bih1, bhh1, Wi1, bi1, Wj1, bj1,
           W2, b2, Wih2, Whh2, bih2, bhh2, Wi2, bi2, Wj2, bj2,
           Wf0, bf0, Wf1, bf1, Wout, bout,
           edge_index_bond, etypes_bond, edge_index_knn, num_atoms_ligand):
    zeros = jnp.zeros((NACC, F), jnp.float32)

    def _pack_idx(gi, di):
        p = jnp.left_shift(di, GSHIFT) | gi
        p = p.reshape(NW, E_PER_W)
        return jnp.pad(p, ((0, 0), (0, EPAD - E_PER_W)),
                       constant_values=NDUMMY << GSHIFT)

    src1, dst1 = edge_index_bond[0], edge_index_bond[1]
    src2, dst2 = edge_index_knn[0], edge_index_knn[1]
    pidx1 = _pack_idx(etypes_bond * N + src1, dst1)
    pidx2 = _pack_idx(src2, dst2)

    W1T = W1.transpose(0, 2, 1)
    W2T0 = W2[:1].transpose(0, 2, 1)
    Wih1T, Whh1T = Wih1.T, Whh1.T
    Wih2T, Whh2T = Wih2.T, Whh2.T
    bih1r, bhh1r = bih1.reshape(1, -1), bhh1.reshape(1, -1)
    bih2r, bhh2r = bih2.reshape(1, -1), bhh2.reshape(1, -1)
    Wi1aT, Wi1bT = Wi1[:, :F].T, Wi1[:, F:].T
    Wi2aT, Wi2bT = Wi2[:, :F].T, Wi2[:, F:].T
    Wj1T, Wj2T = Wj1.T, Wj2.T
    bi1r, bj1r = bi1.reshape(1, F), bj1.reshape(1, F)
    bi2r, bj2r = bi2.reshape(1, F), bj2.reshape(1, F)

    na = jnp.reshape(num_atoms_ligand, (1,)).astype(jnp.int32)

    table = _transform(x, W1T, b1, 12)
    a2 = _gs_call(table, pidx1, zeros)
    h, table = _gru_tf(a2, x, Wih1T, Whh1T, bih1r, bhh1r, W1T, b1, 12, 1000)
    a2 = _gs_call(table, pidx1, zeros)
    h, table = _gru_gate_tf(a2, h, x, Wih1T, Whh1T, bih1r, bhh1r,
                            Wi1aT, Wi1bT, bi1r, Wj1T, bj1r, W2T0, b2[:1])
    feat2 = h

    a2 = _gs_call(table, pidx2, zeros)
    h, table = _gru_tf(a2, h, Wih2T, Whh2T, bih2r, bhh2r, W2T0, b2[:1], 1, BN)
    a2 = _gs_call(table, pidx2, zeros)
    out = _gru_gate_readout(a2, h, feat2, Wih2T, Whh2T, bih2r, bhh2r,
                            Wi2aT, Wi2bT, bi2r, Wj2T, bj2r, na,
                            Wf0.T, bf0.reshape(1, -1), Wf1.T,
                            bf1.reshape(1, -1), Wout.T, bout.reshape(1, 1))
    return out.reshape(1)


# speedup vs baseline: 2.7121x; 2.7121x over previous
import functools

import jax
import jax.numpy as jnp
from jax import lax
from jax.experimental import pallas as pl
from jax.experimental.pallas import tpu as pltpu
from jax.experimental.pallas import tpu_sc as plsc

N = 10000
F = 128
NC, NS = 2, 16
NW = NC * NS
E = 320000
E_PER_W = E // NW
CH = 80
NCHUNK = E_PER_W // CH
NBUF = 2
BN = 2000


def _full(shape):
    zeros = (0,) * len(shape)
    return pl.BlockSpec(shape, lambda *_: zeros)


def _transform_body(h_ref, wt_ref, b_ref, out_ref):
    out_ref[...] = (
        jnp.dot(h_ref[...], wt_ref[0], preferred_element_type=jnp.float32)
        + b_ref[0]
    )


def _transform(h, WT, b, T):
    nb = N // BN
    return pl.pallas_call(
        _transform_body,
        grid=(nb, T),
        in_specs=[
            pl.BlockSpec((BN, F), lambda i, t: (i, 0)),
            pl.BlockSpec((1, F, F), lambda i, t: (t, 0, 0)),
            pl.BlockSpec((1, 1, F), lambda i, t: (t, 0, 0)),
        ],
        out_specs=pl.BlockSpec((BN, F), lambda i, t: (t * (N // BN) + i, 0)),
        out_shape=jax.ShapeDtypeStruct((T * N, F), jnp.float32),
    )(h, WT, b.reshape(T, 1, F))


def _gru_math(a2, h, wihT, whhT, bih, bhh):
    a = a2[0] + a2[1]
    gi = jnp.dot(a, wihT, preferred_element_type=jnp.float32) + bih
    gh = jnp.dot(h, whhT, preferred_element_type=jnp.float32) + bhh
    r = jax.nn.sigmoid(gi[:, :F] + gh[:, :F])
    z = jax.nn.sigmoid(gi[:, F:2 * F] + gh[:, F:2 * F])
    n = jnp.tanh(gi[:, 2 * F:] + r * gh[:, 2 * F:])
    return (1.0 - z) * n + z * h


def _gate_math(h, f, wiaT, wibT, bi, wjT, bj):
    g = (
        jnp.dot(h, wiaT, preferred_element_type=jnp.float32)
        + jnp.dot(f, wibT, preferred_element_type=jnp.float32)
        + bi
    )
    return jax.nn.sigmoid(g) * (
        jnp.dot(h, wjT, preferred_element_type=jnp.float32) + bj
    )


def _gru_tf(a2, h, WihT, WhhT, bih, bhh, WT, b, T, bn):
    nb = N // bn

    def body(a_ref, h_ref, wihT_ref, whhT_ref, bih_ref, bhh_ref, wT_ref,
             b_ref, hout_ref, tout_ref):
        hnew = _gru_math(a_ref[...], h_ref[...], wihT_ref[...], whhT_ref[...],
                         bih_ref[...], bhh_ref[...])
        hout_ref[...] = hnew
        for t in range(T):
            tout_ref[t] = (
                jnp.dot(hnew, wT_ref[t], preferred_element_type=jnp.float32)
                + b_ref[t]
            )

    hout, tout = pl.pallas_call(
        body,
        grid=(nb,),
        in_specs=[
            pl.BlockSpec((2, bn, F), lambda i: (0, i, 0)),
            pl.BlockSpec((bn, F), lambda i: (i, 0)),
            _full((F, 3 * F)),
            _full((F, 3 * F)),
            _full((1, 3 * F)),
            _full((1, 3 * F)),
            _full((T, F, F)),
            _full((T, 1, F)),
        ],
        out_specs=[
            pl.BlockSpec((bn, F), lambda i: (i, 0)),
            pl.BlockSpec((T, bn, F), lambda i: (0, i, 0)),
        ],
        out_shape=[
            jax.ShapeDtypeStruct((N, F), jnp.float32),
            jax.ShapeDtypeStruct((T, N, F), jnp.float32),
        ],
    )(a2, h, WihT, WhhT, bih, bhh, WT, b.reshape(T, 1, F))
    return hout, tout.reshape(T * N, F)


def _gru_gate_tf(a2, h, feat, WihT, WhhT, bih, bhh,
                 WiaT, WibT, bi, WjT, bj, WT, b):
    nb = N // BN

    def body(a_ref, h_ref, f_ref, wihT_ref, whhT_ref, bih_ref, bhh_ref,
             wiaT_ref, wibT_ref, bi_ref, wjT_ref, bj_ref, wT_ref, b_ref,
             hout_ref, tout_ref):
        hnew = _gru_math(a_ref[...], h_ref[...], wihT_ref[...], whhT_ref[...],
                         bih_ref[...], bhh_ref[...])
        hg = _gate_math(hnew, f_ref[...], wiaT_ref[...], wibT_ref[...],
                        bi_ref[...], wjT_ref[...], bj_ref[...])
        hout_ref[...] = hg
        tout_ref[...] = (
            jnp.dot(hg, wT_ref[0], preferred_element_type=jnp.float32)
            + b_ref[0]
        )

    hout, tout = pl.pallas_call(
        body,
        grid=(nb,),
        in_specs=[
            pl.BlockSpec((2, BN, F), lambda i: (0, i, 0)),
            pl.BlockSpec((BN, F), lambda i: (i, 0)),
            pl.BlockSpec((BN, F), lambda i: (i, 0)),
            _full((F, 3 * F)),
            _full((F, 3 * F)),
            _full((1, 3 * F)),
            _full((1, 3 * F)),
            _full((F, F)),
            _full((F, F)),
            _full((1, F)),
            _full((F, F)),
            _full((1, F)),
            _full((1, F, F)),
            _full((1, 1, F)),
        ],
        out_specs=[
            pl.BlockSpec((BN, F), lambda i: (i, 0)),
            pl.BlockSpec((BN, F), lambda i: (i, 0)),
        ],
        out_shape=[
            jax.ShapeDtypeStruct((N, F), jnp.float32),
            jax.ShapeDtypeStruct((N, F), jnp.float32),
        ],
    )(a2, h, feat, WihT, WhhT, bih, bhh, WiaT, WibT, bi, WjT, bj,
      WT, b.reshape(1, 1, F))
    return hout, tout


def _gru_gate_readout(a2, h, feat, WihT, WhhT, bih, bhh,
                      WiaT, WibT, bi, WjT, bj, na,
                      Wf0T, bf0, Wf1T, bf1, WoutT, bout):
    nb = N // BN

    def body(a_ref, h_ref, f_ref, wihT_ref, whhT_ref, bih_ref, bhh_ref,
             wiaT_ref, wibT_ref, bi_ref, wjT_ref, bj_ref, na_ref,
             wf0T_ref, bf0_ref, wf1T_ref, bf1_ref, woutT_ref, bout_ref,
             out_ref, acc_ref):
        i = pl.program_id(0)

        @pl.when(i == 0)
        def _():
            acc_ref[...] = jnp.zeros_like(acc_ref)

        hnew = _gru_math(a_ref[...], h_ref[...], wihT_ref[...], whhT_ref[...],
                         bih_ref[...], bhh_ref[...])
        hg = _gate_math(hnew, f_ref[...], wiaT_ref[...], wibT_ref[...],
                        bi_ref[...], wjT_ref[...], bj_ref[...])
        rows = jax.lax.broadcasted_iota(jnp.int32, (BN, 1), 0) + i * BN
        masked = jnp.where(rows < na_ref[0], hg, 0.0)
        acc_ref[...] += jnp.sum(masked.reshape(BN // 8, 8, F), axis=0)

        @pl.when(i == pl.num_programs(0) - 1)
        def _():
            hp = jax.lax.Precision.HIGHEST
            v = jnp.sum(acc_ref[...], axis=0, keepdims=True)
            v = jax.nn.relu(
                jnp.dot(v, wf0T_ref[...], preferred_element_type=jnp.float32,
                        precision=hp) + bf0_ref[...])
            v = jax.nn.relu(
                jnp.dot(v, wf1T_ref[...], preferred_element_type=jnp.float32,
                        precision=hp) + bf1_ref[...])
            out_ref[...] = (
                jnp.dot(v, woutT_ref[...], preferred_element_type=jnp.float32,
                        precision=hp) + bout_ref[...])

    return pl.pallas_call(
        body,
        grid=(nb,),
        in_specs=[
            pl.BlockSpec((2, BN, F), lambda i: (0, i, 0)),
            pl.BlockSpec((BN, F), lambda i: (i, 0)),
            pl.BlockSpec((BN, F), lambda i: (i, 0)),
            _full((F, 3 * F)),
            _full((F, 3 * F)),
            _full((1, 3 * F)),
            _full((1, 3 * F)),
            _full((F, F)),
            _full((F, F)),
            _full((1, F)),
            _full((F, F)),
            _full((1, F)),
            pl.BlockSpec(memory_space=pltpu.SMEM),
            _full((F, F)),
            _full((1, F)),
            _full((F, F)),
            _full((1, F)),
            _full((F, 1)),
            _full((1, 1)),
        ],
        out_specs=pl.BlockSpec((1, 1), lambda i: (0, 0)),
        out_shape=jax.ShapeDtypeStruct((1, 1), jnp.float32),
        scratch_shapes=[pltpu.VMEM((8, F), jnp.float32)],
    )(a2, h, feat, WihT, WhhT, bih, bhh, WiaT, WibT, bi, WjT, bj, na,
      Wf0T, bf0, Wf1T, bf1, WoutT, bout)


@functools.lru_cache(maxsize=None)
def _make_gs(t_rows):
    mesh = plsc.VectorSubcoreMesh(
        core_axis_name="c", subcore_axis_name="s", num_cores=NC, num_subcores=NS)

    @functools.partial(
        pl.kernel, mesh=mesh,
        out_type=jax.ShapeDtypeStruct((NC, N, F), jnp.float32),
        scratch_types=(
            [
                pltpu.VMEM((E_PER_W,), jnp.int32),
                pltpu.VMEM((NCHUNK, CH), jnp.int32),
                pltpu.VMEM((NBUF, CH, F), jnp.float32),
                pltpu.VMEM_SHARED((N, F), jnp.float32),
            ]
            + [pltpu.SemaphoreType.DMA] * (3 * NBUF)
        ),
    )
    def gs(table, gidx, didx, zeros, out, gi_v, di_v, rows_v, acc, *sems):
        sem_g = sems[:NBUF]
        sem_g2 = sems[NBUF:2 * NBUF]
        sem_s = sems[2 * NBUF:]
        cid = lax.axis_index("c")
        sid = lax.axis_index("s")
        wid = sid * NC + cid
        base = pl.multiple_of(sid * 624, 8)

        @pl.when(sid < NS - 1)
        def _():
            pltpu.sync_copy(zeros.at[pl.ds(base, 624)], acc.at[pl.ds(base, 624)])

        @pl.when(sid == NS - 1)
        def _():
            pltpu.sync_copy(zeros.at[pl.ds(base, 640)], acc.at[pl.ds(base, 640)])

        pltpu.sync_copy(gidx.at[wid], gi_v)
        pltpu.sync_copy(didx.at[wid], di_v)
        plsc.subcore_barrier()

        HH = CH // 2

        def gather_start(b, c):
            pltpu.async_copy(table.at[gi_v.at[pl.ds(c * CH, HH)]],
                             rows_v.at[b, pl.ds(0, HH)], sem_g[b])
            pltpu.async_copy(table.at[gi_v.at[pl.ds(c * CH + HH, HH)]],
                             rows_v.at[b, pl.ds(HH, HH)], sem_g2[b])

        def gather_wait(b, c):
            pltpu.make_async_copy(table.at[gi_v.at[pl.ds(c * CH, HH)]],
                                  rows_v.at[b, pl.ds(0, HH)], sem_g[b]).wait()
            pltpu.make_async_copy(table.at[gi_v.at[pl.ds(c * CH + HH, HH)]],
                                  rows_v.at[b, pl.ds(HH, HH)], sem_g2[b]).wait()

        def scatter_start(b, c):
            pltpu.async_copy(rows_v.at[b], acc.at[di_v.at[c]], sem_s[b],
                             add=True)

        def scatter_wait(b, c):
            pltpu.make_async_copy(rows_v.at[b], acc.at[di_v.at[c]],
                                  sem_s[b]).wait()

        gather_start(0, 0)

        def round_body(r, carry):
            c0 = 2 * r
            c1 = c0 + 1

            @pl.when(r > 0)
            def _():
                scatter_wait(1, c0 - 1)

            gather_start(1, c1)
            gather_wait(0, c0)
            scatter_start(0, c0)

            scatter_wait(0, c0)

            @pl.when(c1 + 1 < NCHUNK)
            def _():
                gather_start(0, c1 + 1)

            gather_wait(1, c1)
            scatter_start(1, c1)
            return carry

        lax.fori_loop(0, NCHUNK // 2, round_body, 0)
        c_last = NCHUNK - 1
        scatter_wait(1, c_last - 1)
        gather_wait(0, c_last)
        scatter_start(0, c_last)
        scatter_wait(0, c_last)
        plsc.subcore_barrier()

        @pl.when(sid < NS - 1)
        def _():
            pltpu.sync_copy(acc.at[pl.ds(base, 624)],
                            out.at[cid, pl.ds(base, 624)])

        @pl.when(sid == NS - 1)
        def _():
            pltpu.sync_copy(acc.at[pl.ds(base, 640)],
                            out.at[cid, pl.ds(base, 640)])

    return gs


def _gs_call(table, gidx, didx, zeros):
    return _make_gs(table.shape[0])(table, gidx, didx, zeros)


def kernel(x, W1, b1, Wih1, Whh1, bih1, bhh1, Wi1, bi1, Wj1, bj1,
           W2, b2, Wih2, Whh2, bih2, bhh2, Wi2, bi2, Wj2, bj2,
           Wf0, bf0, Wf1, bf1, Wout, bout,
           edge_index_bond, etypes_bond, edge_index_knn, num_atoms_ligand):
    zeros = jnp.zeros((N, F), jnp.float32)

    src1, dst1 = edge_index_bond[0], edge_index_bond[1]
    src2, dst2 = edge_index_knn[0], edge_index_knn[1]
    gidx1 = (etypes_bond * N + src1).reshape(NW, E_PER_W)
    didx1 = dst1.reshape(NW, NCHUNK, CH)
    gidx2 = src2.reshape(NW, E_PER_W)
    didx2 = dst2.reshape(NW, NCHUNK, CH)

    W1T = W1.transpose(0, 2, 1)
    W2T0 = W2[:1].transpose(0, 2, 1)
    Wih1T, Whh1T = Wih1.T, Whh1.T
    Wih2T, Whh2T = Wih2.T, Whh2.T
    bih1r, bhh1r = bih1.reshape(1, -1), bhh1.reshape(1, -1)
    bih2r, bhh2r = bih2.reshape(1, -1), bhh2.reshape(1, -1)
    Wi1aT, Wi1bT = Wi1[:, :F].T, Wi1[:, F:].T
    Wi2aT, Wi2bT = Wi2[:, :F].T, Wi2[:, F:].T
    Wj1T, Wj2T = Wj1.T, Wj2.T
    bi1r, bj1r = bi1.reshape(1, F), bj1.reshape(1, F)
    bi2r, bj2r = bi2.reshape(1, F), bj2.reshape(1, F)

    na = jnp.reshape(num_atoms_ligand, (1,)).astype(jnp.int32)

    table = _transform(x, W1T, b1, 12)
    a2 = _gs_call(table, gidx1, didx1, zeros)
    h, table = _gru_tf(a2, x, Wih1T, Whh1T, bih1r, bhh1r, W1T, b1, 12, 1000)
    a2 = _gs_call(table, gidx1, didx1, zeros)
    h, table = _gru_gate_tf(a2, h, x, Wih1T, Whh1T, bih1r, bhh1r,
                            Wi1aT, Wi1bT, bi1r, Wj1T, bj1r, W2T0, b2[:1])
    feat2 = h

    a2 = _gs_call(table, gidx2, didx2, zeros)
    h, table = _gru_tf(a2, h, Wih2T, Whh2T, bih2r, bhh2r, W2T0, b2[:1], 1, BN)
    a2 = _gs_call(table, gidx2, didx2, zeros)
    out = _gru_gate_readout(a2, h, feat2, Wih2T, Whh2T, bih2r, bhh2r,
                            Wi2aT, Wi2bT, bi2r, Wj2T, bj2r, na,
                            Wf0.T, bf0.reshape(1, -1), Wf1.T,
                            bf1.reshape(1, -1), Wout.T, bout.reshape(1, 1))
    return out.reshape(1)
